# baseline (speedup 1.0000x reference)
import functools

import jax
import jax.numpy as jnp
from jax import lax
from jax.experimental import pallas as pl
from jax.experimental.pallas import tpu as pltpu
from jax.experimental.pallas import tpu_sc as plsc

N = 10000
NP_ = 10240
E = 320000
D = 128
G = 16
NC = 2
NS = 16
NW = NC * NS
CK = 80
CH = 128
EPT = CH * CK
E_PAD = NW * EPT
RPT = NP_ // NS
RZB = 40
SB = 16
NF = D // 16

BN_BLK = 1024
NB = NP_ // BN_BLK


def _sc_solo_passes(num_srcs, num_idx, passes):
    np_ = len(passes)
    mesh = plsc.VectorSubcoreMesh(core_axis_name="c", subcore_axis_name="s",
                                  num_cores=NC, num_subcores=NS)

    def body(*refs):
        srcs = refs[:num_srcs]
        sidx = refs[num_srcs:num_srcs + num_idx]
        didx = refs[num_srcs + num_idx:num_srcs + 2 * num_idx]
        attr = refs[num_srcs + 2 * num_idx:num_srcs + 3 * num_idx]
        out = refs[num_srcs + 3 * num_idx]
        (acc, sidx_v, didx_v, attr_v, rows0, rows1, zbuf_v,
         sem0, sem1, ssem0, ssem1) = refs[num_srcs + 3 * num_idx + 1:]
        rows = (rows0, rows1)
        sems = (sem0, sem1)
        ssems = (ssem0, ssem1)
        c = lax.axis_index("c")
        s = lax.axis_index("s")
        base = s * RPT

        def zrow(r, carry):
            for f in range(NF):
                zbuf_v[r, pl.ds(f * 16, 16)] = jnp.zeros((16,), jnp.float32)
            return carry
        lax.fori_loop(0, RZB, zrow, 0)

        for p, (si, ii) in enumerate(passes):
            for k in range(RPT // RZB):
                pltpu.sync_copy(zbuf_v, acc.at[pl.ds(base + k * RZB, RZB), :])
            plsc.subcore_barrier()
            src = srcs[si]

            dn = lax.GatherDimensionNumbers(
                offset_dims=(), collapsed_slice_dims=(0,), start_index_map=(0,))

            def scale_rows(rv, j):
                @plsc.parallel_loop(0, CK // 16, 1, unroll=2)
                def _group16(gg):
                    a16 = attr_v[pl.ds(j * CK + gg * 16, 16)]
                    for k in range(16):
                        av = lax.gather(
                            a16, jnp.full((16, 1), k, jnp.int32), dn, (1,),
                            mode=lax.GatherScatterMode.PROMISE_IN_BOUNDS)
                        e = gg * 16 + k
                        for f in range(NF):
                            rv[e, pl.ds(f * 16, 16)] = (
                                rv[e, pl.ds(f * 16, 16)] * av)

            NG = SB // 2

            def block(b, carry, si=si, ii=ii):
                pltpu.sync_copy(sidx[ii].at[c, s, pl.ds(b * SB, SB)], sidx_v)
                pltpu.sync_copy(didx[ii].at[c, s, pl.ds(b * SB, SB)], didx_v)
                pltpu.sync_copy(attr[ii].at[c, s, pl.ds(b * SB * CK, SB * CK)],
                                attr_v)
                for t in range(2):
                    pltpu.async_copy(srcs[si].at[sidx_v.at[t]], rows[t], sems[t])

                def group(g, carry2):
                    for t in range(2):
                        j = g * 2 + t
                        pltpu.make_async_copy(
                            srcs[si].at[sidx_v.at[j]], rows[t], sems[t]).wait()
                        scale_rows(rows[t], j)
                        pltpu.async_copy(rows[t], acc.at[didx_v.at[j]],
                                         ssems[t], add=True)

                        @pl.when(g < NG - 1)
                        def _prefetch(t=t, j=j):
                            pltpu.make_async_copy(
                                rows[t], acc.at[didx_v.at[j]], ssems[t]).wait()
                            pltpu.async_copy(
                                srcs[si].at[sidx_v.at[j + 2]], rows[t], sems[t])
                    return carry2
                lax.fori_loop(0, NG, group, 0)
                for t in range(2):
                    pltpu.make_async_copy(
                        rows[t], acc.at[didx_v.at[SB - 2 + t]], ssems[t]).wait()
                return carry
            lax.fori_loop(0, CH // SB, block, 0)
            plsc.subcore_barrier()
            pltpu.sync_copy(acc.at[pl.ds(base, RPT), :],
                            out.at[p, c, pl.ds(base, RPT), :])

    return pl.kernel(
        body,
        out_type=jax.ShapeDtypeStruct((np_, NC, NP_, D), jnp.float32),
        mesh=mesh,
        scratch_types=[
            pltpu.VMEM_SHARED((NP_, D), jnp.float32),
            pltpu.VMEM((SB, CK), jnp.int32),
            pltpu.VMEM((SB, CK), jnp.int32),
            pltpu.VMEM((SB * CK,), jnp.float32),
            pltpu.VMEM((CK, D), jnp.float32),
            pltpu.VMEM((CK, D), jnp.float32),
            pltpu.VMEM((RZB, D), jnp.float32),
            pltpu.SemaphoreType.DMA,
            pltpu.SemaphoreType.DMA,
            pltpu.SemaphoreType.DMA,
            pltpu.SemaphoreType.DMA,
        ],
    )


def _tc_combine(pa):
    def body(pa_ref, xh1, xh2, xh3, y0):
        for i, ref in enumerate((xh1, xh2, xh3)):
            ref[...] = jnp.abs(pa_ref[i, 0] + pa_ref[i, 1])
        y0[...] = pa_ref[3, 0] + pa_ref[3, 1]

    row = jax.ShapeDtypeStruct((NP_, D), jnp.float32)
    return pl.pallas_call(
        body,
        grid=(NB,),
        in_specs=[pl.BlockSpec((4, NC, BN_BLK, D), lambda i: (0, 0, i, 0))],
        out_specs=[pl.BlockSpec((BN_BLK, D), lambda i: (i, 0))] * 4,
        out_shape=[row, row, row, row],
    )(pa)


def _elu(v):
    return jnp.where(v > 0, v, jnp.exp(jnp.minimum(v, 0.0)) - 1.0)


def _tc_final(x, y0, qb, batch_r, mlp_W, mlp_vec, fc1_W, fc1_vec, fc2_W, fc2_vec):
    C = fc2_W.shape[1]
    H = fc1_W.shape[1]
    EPS = 1e-5

    def body(x_ref, y0_ref, qb_ref, b_ref, mW_ref, mv_ref, f1W_ref, f1v_ref,
             f2W_ref, f2v_ref, out_ref, acc, cnt):
        i = pl.program_id(0)

        @pl.when(i == 0)
        def _init():
            acc[...] = jnp.zeros_like(acc)
            cnt[...] = jnp.zeros_like(cnt)

        b = b_ref[0, 0, :]
        oh = (b[:, None] == lax.broadcasted_iota(jnp.int32, (BN_BLK, G), 1)
              ).astype(jnp.float32)

        def segdot(z):
            return lax.dot_general(oh, z, (((0,), (0,)), ((), ())),
                                   preferred_element_type=jnp.float32)

        acc[0] += segdot(x_ref[...])
        acc[1] += segdot(_elu(y0_ref[...]))
        for t in range(3):
            y = qb_ref[t, 0] + qb_ref[t, 1]
            acc[2 + t] += segdot(_elu(y))
        cnt[0, :] += jnp.sum(oh, axis=0)

        @pl.when(i == NB - 1)
        def _tail():
            h = acc[0]
            csum = jnp.zeros((D,), jnp.float32)
            for t in range(4):
                bvec, gam, bet, mean, var = (mv_ref[k, t] for k in range(5))
                sc = gam * lax.rsqrt(var + EPS)
                h = h + lax.dot_general(
                    acc[1 + t], mW_ref[t] * sc[None, :],
                    (((1,), (0,)), ((), ())), preferred_element_type=jnp.float32)
                csum = csum + (bvec - mean) * sc + bet
            h = h + cnt[0, :G][:, None] * csum[None, :]
            b1, g1, be1, m1, v1 = (f1v_ref[k] for k in range(5))
            s1 = g1 * lax.rsqrt(v1 + EPS)
            h1 = lax.dot_general(h, f1W_ref[...], (((1,), (0,)), ((), ())),
                                 preferred_element_type=jnp.float32)
            h1 = (h1 + b1[None, :] - m1[None, :]) * s1[None, :] + be1[None, :]
            h1 = jnp.maximum(h1, 0.0)
            b2, g2, be2, m2, v2 = (f2v_ref[k] for k in range(5))
            s2 = g2 * lax.rsqrt(v2 + EPS)
            o = lax.dot_general(h1, f2W_ref[...], (((1,), (0,)), ((), ())),
                                preferred_element_type=jnp.float32)
            out_ref[...] = (o + b2[None, :] - m2[None, :]) * s2[None, :] + be2[None, :]

    full = lambda shape: pl.BlockSpec(shape, lambda i: tuple(0 for _ in shape))
    return pl.pallas_call(
        body,
        grid=(NB,),
        in_specs=[
            pl.BlockSpec((BN_BLK, D), lambda i: (i, 0)),
            pl.BlockSpec((BN_BLK, D), lambda i: (i, 0)),
            pl.BlockSpec((3, NC, BN_BLK, D), lambda i: (0, 0, i, 0)),
            pl.BlockSpec((1, 1, BN_BLK), lambda i: (i, 0, 0)),
            full((4, D, D)),
            full((5, 4, D)),
            full((D, H)),
            full((5, H)),
            full((H, C)),
            full((5, C)),
        ],
        out_specs=pl.BlockSpec((G, C), lambda i: (0, 0)),
        out_shape=jax.ShapeDtypeStruct((G, C), jnp.float32),
        scratch_shapes=[
            pltpu.VMEM((5, G, D), jnp.float32),
            pltpu.VMEM((1, G), jnp.float32),
        ],
    )(x, y0, qb, batch_r, mlp_W, mlp_vec, fc1_W, fc1_vec, fc2_W, fc2_vec)


@functools.partial(jax.jit, static_argnums=())
def kernel(x, edge_index, scatter_edge_index_0, scatter_edge_attr_0,
           scatter_edge_index_1, scatter_edge_attr_1, scatter_edge_index_2,
           scatter_edge_attr_2, scatter_edge_index_3, scatter_edge_attr_3,
           batch, mlp_W, mlp_b, mlp_bn_gamma, mlp_bn_beta, mlp_bn_mean,
           mlp_bn_var, fc1_W, fc1_b, fc1_bn_gamma, fc1_bn_beta, fc1_bn_mean,
           fc1_bn_var, fc2_W, fc2_b, fc2_bn_gamma, fc2_bn_beta, fc2_bn_mean,
           fc2_bn_var):
    del edge_index
    xp = jnp.pad(x, ((0, NP_ - N), (0, 0)))
    pad_i = ((jnp.arange(E_PAD - E, dtype=jnp.int32) * 37) % N)
    pad_e = lambda a: jnp.concatenate([a.astype(jnp.int32), pad_i])
    r4 = lambda a: pad_e(a).reshape(NC, NS, CH, CK)
    idx = [scatter_edge_index_1, scatter_edge_index_2, scatter_edge_index_3,
           scatter_edge_index_0]
    att = [scatter_edge_attr_1, scatter_edge_attr_2, scatter_edge_attr_3,
           scatter_edge_attr_0]
    sidx = [r4(a[0]) for a in idx]
    didx = [r4(a[1]) for a in idx]
    zpad = jnp.zeros((E_PAD - E,), jnp.float32)
    attr = [jnp.concatenate([a, zpad]).reshape(NC, NS, EPT) for a in att]

    sc_a = _sc_solo_passes(1, 4, ((0, 0), (0, 1), (0, 2), (0, 3)))
    pa = sc_a(xp, *sidx, *didx, *attr)

    xh1, xh2, xh3, y0 = _tc_combine(pa)

    sc_b = _sc_solo_passes(3, 1, ((0, 0), (1, 0), (2, 0)))
    qb = sc_b(xh1, xh2, xh3, sidx[3], didx[3], attr[3])

    batch_p = jnp.pad(batch.astype(jnp.int32), (0, NP_ - N),
                      constant_values=-1)
    batch_r = batch_p.reshape(NB, 1, BN_BLK)
    mlp_vec = jnp.stack([mlp_b, mlp_bn_gamma, mlp_bn_beta, mlp_bn_mean, mlp_bn_var])
    fc1_vec = jnp.stack([fc1_b, fc1_bn_gamma, fc1_bn_beta, fc1_bn_mean, fc1_bn_var])
    fc2_vec = jnp.stack([fc2_b, fc2_bn_gamma, fc2_bn_beta, fc2_bn_mean, fc2_bn_var])
    return _tc_final(xp, y0, qb, batch_r, mlp_W, mlp_vec, fc1_W, fc1_vec,
                     fc2_W, fc2_vec)


# speedup vs baseline: 1.1060x; 1.1060x over previous
import functools

import jax
import jax.numpy as jnp
from jax import lax
from jax.experimental import pallas as pl
from jax.experimental.pallas import tpu as pltpu
from jax.experimental.pallas import tpu_sc as plsc

N = 10000
NP_ = 10240
E = 320000
D = 128
G = 16
NC = 2
NS = 16
NW = NC * NS
CK = 80
CH = 128
EPT = CH * CK
E_PAD = NW * EPT
RPT = NP_ // NS
RZB = 16
SB = 16
NF = D // 16

BN_BLK = 1024
NB = NP_ // BN_BLK


def _sc_solo_passes(num_srcs, num_idx, passes):
    np_ = len(passes)
    mesh = plsc.VectorSubcoreMesh(core_axis_name="c", subcore_axis_name="s",
                                  num_cores=NC, num_subcores=NS)

    def body(*refs):
        srcs = refs[:num_srcs]
        sidx = refs[num_srcs:num_srcs + num_idx]
        didx = refs[num_srcs + num_idx:num_srcs + 2 * num_idx]
        attr = refs[num_srcs + 2 * num_idx:num_srcs + 3 * num_idx]
        out = refs[num_srcs + 3 * num_idx]
        (acc, sidx_v, didx_v, attr_v, gbuf0, gbuf1, sbuf0, sbuf1, zbuf_v,
         gsem0, gsem1, ssem0, ssem1) = refs[num_srcs + 3 * num_idx + 1:]
        gbuf = (gbuf0, gbuf1)
        sbuf = (sbuf0, sbuf1)
        gsems = (gsem0, gsem1)
        ssems = (ssem0, ssem1)
        c = lax.axis_index("c")
        s = lax.axis_index("s")
        base = s * RPT

        def zrow(r, carry):
            for f in range(NF):
                zbuf_v[r, pl.ds(f * 16, 16)] = jnp.zeros((16,), jnp.float32)
            return carry
        lax.fori_loop(0, RZB, zrow, 0)

        for p, (si, ii) in enumerate(passes):
            for k in range(RPT // RZB):
                pltpu.sync_copy(zbuf_v, acc.at[pl.ds(base + k * RZB, RZB), :])
            plsc.subcore_barrier()
            src = srcs[si]

            dn = lax.GatherDimensionNumbers(
                offset_dims=(), collapsed_slice_dims=(0,), start_index_map=(0,))

            def scale_rows(gv, sv, j):
                @plsc.parallel_loop(0, CK // 16, 1, unroll=2)
                def _group16(gg):
                    a16 = attr_v[pl.ds(j * CK + gg * 16, 16)]
                    for k in range(16):
                        av = lax.gather(
                            a16, jnp.full((16, 1), k, jnp.int32), dn, (1,),
                            mode=lax.GatherScatterMode.PROMISE_IN_BOUNDS)
                        e = gg * 16 + k
                        for f in range(NF):
                            sv[e, pl.ds(f * 16, 16)] = (
                                gv[e, pl.ds(f * 16, 16)] * av)

            NG = SB // 2

            def block(b, carry, si=si, ii=ii):
                pltpu.sync_copy(sidx[ii].at[c, s, pl.ds(b * SB, SB)], sidx_v)
                pltpu.sync_copy(didx[ii].at[c, s, pl.ds(b * SB, SB)], didx_v)
                pltpu.sync_copy(attr[ii].at[c, s, pl.ds(b * SB * CK, SB * CK)],
                                attr_v)
                for t in range(2):
                    pltpu.async_copy(srcs[si].at[sidx_v.at[t]], gbuf[t], gsems[t])

                def group(g, carry2):
                    for t in range(2):
                        j = g * 2 + t
                        pltpu.make_async_copy(
                            srcs[si].at[sidx_v.at[j]], gbuf[t], gsems[t]).wait()

                        @pl.when(g > 0)
                        def _drain(t=t, j=j):
                            pltpu.make_async_copy(
                                sbuf[t], acc.at[didx_v.at[j]], ssems[t]).wait()
                        scale_rows(gbuf[t], sbuf[t], j)

                        @pl.when(g < NG - 1)
                        def _prefetch(t=t, j=j):
                            pltpu.async_copy(
                                srcs[si].at[sidx_v.at[j + 2]], gbuf[t], gsems[t])
                        pltpu.async_copy(sbuf[t], acc.at[didx_v.at[j]],
                                         ssems[t], add=True)
                    return carry2
                lax.fori_loop(0, NG, group, 0)
                for t in range(2):
                    pltpu.make_async_copy(
                        sbuf[t], acc.at[didx_v.at[SB - 2 + t]], ssems[t]).wait()
                return carry
            lax.fori_loop(0, CH // SB, block, 0)
            plsc.subcore_barrier()
            pltpu.sync_copy(acc.at[pl.ds(base, RPT), :],
                            out.at[p, c, pl.ds(base, RPT), :])

    return pl.kernel(
        body,
        out_type=jax.ShapeDtypeStruct((np_, NC, NP_, D), jnp.float32),
        mesh=mesh,
        scratch_types=[
            pltpu.VMEM_SHARED((NP_, D), jnp.float32),
            pltpu.VMEM((SB, CK), jnp.int32),
            pltpu.VMEM((SB, CK), jnp.int32),
            pltpu.VMEM((SB * CK,), jnp.float32),
            pltpu.VMEM((CK, D), jnp.float32),
            pltpu.VMEM((CK, D), jnp.float32),
            pltpu.VMEM((CK, D), jnp.float32),
            pltpu.VMEM((CK, D), jnp.float32),
            pltpu.VMEM((RZB, D), jnp.float32),
            pltpu.SemaphoreType.DMA,
            pltpu.SemaphoreType.DMA,
            pltpu.SemaphoreType.DMA,
            pltpu.SemaphoreType.DMA,
        ],
    )


def _tc_combine(pa):
    def body(pa_ref, xh1, xh2, xh3, y0):
        for i, ref in enumerate((xh1, xh2, xh3)):
            ref[...] = jnp.abs(pa_ref[i, 0] + pa_ref[i, 1])
        y0[...] = pa_ref[3, 0] + pa_ref[3, 1]

    row = jax.ShapeDtypeStruct((NP_, D), jnp.float32)
    return pl.pallas_call(
        body,
        grid=(NB,),
        in_specs=[pl.BlockSpec((4, NC, BN_BLK, D), lambda i: (0, 0, i, 0))],
        out_specs=[pl.BlockSpec((BN_BLK, D), lambda i: (i, 0))] * 4,
        out_shape=[row, row, row, row],
    )(pa)


def _elu(v):
    return jnp.where(v > 0, v, jnp.exp(jnp.minimum(v, 0.0)) - 1.0)


def _tc_final(x, y0, qb, batch_r, mlp_W, mlp_vec, fc1_W, fc1_vec, fc2_W, fc2_vec):
    C = fc2_W.shape[1]
    H = fc1_W.shape[1]
    EPS = 1e-5

    def body(x_ref, y0_ref, qb_ref, b_ref, mW_ref, mv_ref, f1W_ref, f1v_ref,
             f2W_ref, f2v_ref, out_ref, acc, cnt):
        i = pl.program_id(0)

        @pl.when(i == 0)
        def _init():
            acc[...] = jnp.zeros_like(acc)
            cnt[...] = jnp.zeros_like(cnt)

        b = b_ref[0, 0, :]
        oh = (b[:, None] == lax.broadcasted_iota(jnp.int32, (BN_BLK, G), 1)
              ).astype(jnp.float32)

        def segdot(z):
            return lax.dot_general(oh, z, (((0,), (0,)), ((), ())),
                                   preferred_element_type=jnp.float32)

        acc[0] += segdot(x_ref[...])
        acc[1] += segdot(_elu(y0_ref[...]))
        for t in range(3):
            y = qb_ref[t, 0] + qb_ref[t, 1]
            acc[2 + t] += segdot(_elu(y))
        cnt[0, :] += jnp.sum(oh, axis=0)

        @pl.when(i == NB - 1)
        def _tail():
            h = acc[0]
            csum = jnp.zeros((D,), jnp.float32)
            for t in range(4):
                bvec, gam, bet, mean, var = (mv_ref[k, t] for k in range(5))
                sc = gam * lax.rsqrt(var + EPS)
                h = h + lax.dot_general(
                    acc[1 + t], mW_ref[t] * sc[None, :],
                    (((1,), (0,)), ((), ())), preferred_element_type=jnp.float32)
                csum = csum + (bvec - mean) * sc + bet
            h = h + cnt[0, :G][:, None] * csum[None, :]
            b1, g1, be1, m1, v1 = (f1v_ref[k] for k in range(5))
            s1 = g1 * lax.rsqrt(v1 + EPS)
            h1 = lax.dot_general(h, f1W_ref[...], (((1,), (0,)), ((), ())),
                                 preferred_element_type=jnp.float32)
            h1 = (h1 + b1[None, :] - m1[None, :]) * s1[None, :] + be1[None, :]
            h1 = jnp.maximum(h1, 0.0)
            b2, g2, be2, m2, v2 = (f2v_ref[k] for k in range(5))
            s2 = g2 * lax.rsqrt(v2 + EPS)
            o = lax.dot_general(h1, f2W_ref[...], (((1,), (0,)), ((), ())),
                                preferred_element_type=jnp.float32)
            out_ref[...] = (o + b2[None, :] - m2[None, :]) * s2[None, :] + be2[None, :]

    full = lambda shape: pl.BlockSpec(shape, lambda i: tuple(0 for _ in shape))
    return pl.pallas_call(
        body,
        grid=(NB,),
        in_specs=[
            pl.BlockSpec((BN_BLK, D), lambda i: (i, 0)),
            pl.BlockSpec((BN_BLK, D), lambda i: (i, 0)),
            pl.BlockSpec((3, NC, BN_BLK, D), lambda i: (0, 0, i, 0)),
            pl.BlockSpec((1, 1, BN_BLK), lambda i: (i, 0, 0)),
            full((4, D, D)),
            full((5, 4, D)),
            full((D, H)),
            full((5, H)),
            full((H, C)),
            full((5, C)),
        ],
        out_specs=pl.BlockSpec((G, C), lambda i: (0, 0)),
        out_shape=jax.ShapeDtypeStruct((G, C), jnp.float32),
        scratch_shapes=[
            pltpu.VMEM((5, G, D), jnp.float32),
            pltpu.VMEM((1, G), jnp.float32),
        ],
    )(x, y0, qb, batch_r, mlp_W, mlp_vec, fc1_W, fc1_vec, fc2_W, fc2_vec)


@functools.partial(jax.jit, static_argnums=())
def kernel(x, edge_index, scatter_edge_index_0, scatter_edge_attr_0,
           scatter_edge_index_1, scatter_edge_attr_1, scatter_edge_index_2,
           scatter_edge_attr_2, scatter_edge_index_3, scatter_edge_attr_3,
           batch, mlp_W, mlp_b, mlp_bn_gamma, mlp_bn_beta, mlp_bn_mean,
           mlp_bn_var, fc1_W, fc1_b, fc1_bn_gamma, fc1_bn_beta, fc1_bn_mean,
           fc1_bn_var, fc2_W, fc2_b, fc2_bn_gamma, fc2_bn_beta, fc2_bn_mean,
           fc2_bn_var):
    del edge_index
    xp = jnp.pad(x, ((0, NP_ - N), (0, 0)))
    pad_i = ((jnp.arange(E_PAD - E, dtype=jnp.int32) * 37) % N)
    pad_e = lambda a: jnp.concatenate([a.astype(jnp.int32), pad_i])
    r4 = lambda a: pad_e(a).reshape(NC, NS, CH, CK)
    idx = [scatter_edge_index_1, scatter_edge_index_2, scatter_edge_index_3,
           scatter_edge_index_0]
    att = [scatter_edge_attr_1, scatter_edge_attr_2, scatter_edge_attr_3,
           scatter_edge_attr_0]
    sidx = [r4(a[0]) for a in idx]
    didx = [r4(a[1]) for a in idx]
    zpad = jnp.zeros((E_PAD - E,), jnp.float32)
    attr = [jnp.concatenate([a, zpad]).reshape(NC, NS, EPT) for a in att]

    sc_a = _sc_solo_passes(1, 4, ((0, 0), (0, 1), (0, 2), (0, 3)))
    pa = sc_a(xp, *sidx, *didx, *attr)

    xh1, xh2, xh3, y0 = _tc_combine(pa)

    sc_b = _sc_solo_passes(3, 1, ((0, 0), (1, 0), (2, 0)))
    qb = sc_b(xh1, xh2, xh3, sidx[3], didx[3], attr[3])

    batch_p = jnp.pad(batch.astype(jnp.int32), (0, NP_ - N),
                      constant_values=-1)
    batch_r = batch_p.reshape(NB, 1, BN_BLK)
    mlp_vec = jnp.stack([mlp_b, mlp_bn_gamma, mlp_bn_beta, mlp_bn_mean, mlp_bn_var])
    fc1_vec = jnp.stack([fc1_b, fc1_bn_gamma, fc1_bn_beta, fc1_bn_mean, fc1_bn_var])
    fc2_vec = jnp.stack([fc2_b, fc2_bn_gamma, fc2_bn_beta, fc2_bn_mean, fc2_bn_var])
    return _tc_final(xp, y0, qb, batch_r, mlp_W, mlp_vec, fc1_W, fc1_vec,
                     fc2_W, fc2_vec)


# speedup vs baseline: 1.1198x; 1.0126x over previous
import functools

import jax
import jax.numpy as jnp
from jax import lax
from jax.experimental import pallas as pl
from jax.experimental.pallas import tpu as pltpu
from jax.experimental.pallas import tpu_sc as plsc

N = 10000
NP_ = 10240
E = 320000
D = 128
G = 16
NC = 2
NS = 16
NW = NC * NS
CK = 80
CH = 128
EPT = CH * CK
E_PAD = NW * EPT
RPT = NP_ // NS
RZB = 16
SB = 16
NF = D // 16

BN_BLK = 1024
NB = NP_ // BN_BLK


def _sc_solo_passes(num_srcs, num_idx, passes):
    np_ = len(passes)
    mesh = plsc.VectorSubcoreMesh(core_axis_name="c", subcore_axis_name="s",
                                  num_cores=NC, num_subcores=NS)

    def body(*refs):
        srcs = refs[:num_srcs]
        sidx = refs[num_srcs:num_srcs + num_idx]
        didx = refs[num_srcs + num_idx:num_srcs + 2 * num_idx]
        attr = refs[num_srcs + 2 * num_idx:num_srcs + 3 * num_idx]
        out = refs[num_srcs + 3 * num_idx]
        (acc, sidx_v, didx_v, attr_v, gbuf0, gbuf1, sbuf0, sbuf1, zbuf_v,
         gsem0, gsem1, ssem0, ssem1) = refs[num_srcs + 3 * num_idx + 1:]
        gbuf = (gbuf0, gbuf1)
        sbuf = (sbuf0, sbuf1)
        gsems = (gsem0, gsem1)
        ssems = (ssem0, ssem1)
        c = lax.axis_index("c")
        s = lax.axis_index("s")
        base = s * RPT

        def zrow(r, carry):
            for f in range(NF):
                zbuf_v[r, pl.ds(f * 16, 16)] = jnp.zeros((16,), jnp.float32)
            return carry
        lax.fori_loop(0, RZB, zrow, 0)

        for p, (si, ii) in enumerate(passes):
            for k in range(RPT // RZB):
                pltpu.async_copy(
                    zbuf_v, acc.at[pl.ds(base + k * RZB, RZB), :], ssems[0])
            for k in range(RPT // RZB):
                pltpu.make_async_copy(
                    zbuf_v, acc.at[pl.ds(base + k * RZB, RZB), :],
                    ssems[0]).wait()
            plsc.subcore_barrier()
            src = srcs[si]

            dn = lax.GatherDimensionNumbers(
                offset_dims=(), collapsed_slice_dims=(0,), start_index_map=(0,))

            def scale_rows(gv, sv, j):
                @plsc.parallel_loop(0, CK // 16, 1, unroll=2)
                def _group16(gg):
                    a16 = attr_v[pl.ds(j * CK + gg * 16, 16)]
                    for k in range(16):
                        av = lax.gather(
                            a16, jnp.full((16, 1), k, jnp.int32), dn, (1,),
                            mode=lax.GatherScatterMode.PROMISE_IN_BOUNDS)
                        e = gg * 16 + k
                        for f in range(NF):
                            sv[e, pl.ds(f * 16, 16)] = (
                                gv[e, pl.ds(f * 16, 16)] * av)

            NG = SB // 2

            def block(b, carry, si=si, ii=ii):
                pltpu.sync_copy(sidx[ii].at[c, s, pl.ds(b * SB, SB)], sidx_v)
                pltpu.sync_copy(didx[ii].at[c, s, pl.ds(b * SB, SB)], didx_v)
                pltpu.sync_copy(attr[ii].at[c, s, pl.ds(b * SB * CK, SB * CK)],
                                attr_v)
                for t in range(2):
                    pltpu.async_copy(srcs[si].at[sidx_v.at[t]], gbuf[t], gsems[t])

                def group(g, carry2):
                    for t in range(2):
                        j = g * 2 + t
                        pltpu.make_async_copy(
                            srcs[si].at[sidx_v.at[j]], gbuf[t], gsems[t]).wait()

                        @pl.when(g > 0)
                        def _drain(t=t, j=j):
                            pltpu.make_async_copy(
                                sbuf[t], acc.at[didx_v.at[j]], ssems[t]).wait()
                        scale_rows(gbuf[t], sbuf[t], j)

                        @pl.when(g < NG - 1)
                        def _prefetch(t=t, j=j):
                            pltpu.async_copy(
                                srcs[si].at[sidx_v.at[j + 2]], gbuf[t], gsems[t])
                        pltpu.async_copy(sbuf[t], acc.at[didx_v.at[j]],
                                         ssems[t], add=True)
                    return carry2
                lax.fori_loop(0, NG, group, 0)
                for t in range(2):
                    pltpu.make_async_copy(
                        sbuf[t], acc.at[didx_v.at[SB - 2 + t]], ssems[t]).wait()
                return carry
            lax.fori_loop(0, CH // SB, block, 0)
            plsc.subcore_barrier()
            pltpu.sync_copy(acc.at[pl.ds(base, RPT), :],
                            out.at[p, c, pl.ds(base, RPT), :])

    return pl.kernel(
        body,
        out_type=jax.ShapeDtypeStruct((np_, NC, NP_, D), jnp.float32),
        mesh=mesh,
        scratch_types=[
            pltpu.VMEM_SHARED((NP_, D), jnp.float32),
            pltpu.VMEM((SB, CK), jnp.int32),
            pltpu.VMEM((SB, CK), jnp.int32),
            pltpu.VMEM((SB * CK,), jnp.float32),
            pltpu.VMEM((CK, D), jnp.float32),
            pltpu.VMEM((CK, D), jnp.float32),
            pltpu.VMEM((CK, D), jnp.float32),
            pltpu.VMEM((CK, D), jnp.float32),
            pltpu.VMEM((RZB, D), jnp.float32),
            pltpu.SemaphoreType.DMA,
            pltpu.SemaphoreType.DMA,
            pltpu.SemaphoreType.DMA,
            pltpu.SemaphoreType.DMA,
        ],
    )


def _tc_combine(pa):
    def body(pa_ref, xh1, xh2, xh3, y0):
        for i, ref in enumerate((xh1, xh2, xh3)):
            ref[...] = jnp.abs(pa_ref[i, 0] + pa_ref[i, 1])
        y0[...] = pa_ref[3, 0] + pa_ref[3, 1]

    row = jax.ShapeDtypeStruct((NP_, D), jnp.float32)
    return pl.pallas_call(
        body,
        grid=(NB,),
        in_specs=[pl.BlockSpec((4, NC, BN_BLK, D), lambda i: (0, 0, i, 0))],
        out_specs=[pl.BlockSpec((BN_BLK, D), lambda i: (i, 0))] * 4,
        out_shape=[row, row, row, row],
    )(pa)


def _elu(v):
    return jnp.where(v > 0, v, jnp.exp(jnp.minimum(v, 0.0)) - 1.0)


def _tc_final(x, y0, qb, batch_r, mlp_W, mlp_vec, fc1_W, fc1_vec, fc2_W, fc2_vec):
    C = fc2_W.shape[1]
    H = fc1_W.shape[1]
    EPS = 1e-5

    def body(x_ref, y0_ref, qb_ref, b_ref, mW_ref, mv_ref, f1W_ref, f1v_ref,
             f2W_ref, f2v_ref, out_ref, acc, cnt):
        i = pl.program_id(0)

        @pl.when(i == 0)
        def _init():
            acc[...] = jnp.zeros_like(acc)
            cnt[...] = jnp.zeros_like(cnt)

        b = b_ref[0, 0, :]
        oh = (b[:, None] == lax.broadcasted_iota(jnp.int32, (BN_BLK, G), 1)
              ).astype(jnp.float32)

        def segdot(z):
            return lax.dot_general(oh, z, (((0,), (0,)), ((), ())),
                                   preferred_element_type=jnp.float32)

        acc[0] += segdot(x_ref[...])
        acc[1] += segdot(_elu(y0_ref[...]))
        for t in range(3):
            y = qb_ref[t, 0] + qb_ref[t, 1]
            acc[2 + t] += segdot(_elu(y))
        cnt[0, :] += jnp.sum(oh, axis=0)

        @pl.when(i == NB - 1)
        def _tail():
            h = acc[0]
            csum = jnp.zeros((D,), jnp.float32)
            for t in range(4):
                bvec, gam, bet, mean, var = (mv_ref[k, t] for k in range(5))
                sc = gam * lax.rsqrt(var + EPS)
                h = h + lax.dot_general(
                    acc[1 + t], mW_ref[t] * sc[None, :],
                    (((1,), (0,)), ((), ())), preferred_element_type=jnp.float32)
                csum = csum + (bvec - mean) * sc + bet
            h = h + cnt[0, :G][:, None] * csum[None, :]
            b1, g1, be1, m1, v1 = (f1v_ref[k] for k in range(5))
            s1 = g1 * lax.rsqrt(v1 + EPS)
            h1 = lax.dot_general(h, f1W_ref[...], (((1,), (0,)), ((), ())),
                                 preferred_element_type=jnp.float32)
            h1 = (h1 + b1[None, :] - m1[None, :]) * s1[None, :] + be1[None, :]
            h1 = jnp.maximum(h1, 0.0)
            b2, g2, be2, m2, v2 = (f2v_ref[k] for k in range(5))
            s2 = g2 * lax.rsqrt(v2 + EPS)
            o = lax.dot_general(h1, f2W_ref[...], (((1,), (0,)), ((), ())),
                                preferred_element_type=jnp.float32)
            out_ref[...] = (o + b2[None, :] - m2[None, :]) * s2[None, :] + be2[None, :]

    full = lambda shape: pl.BlockSpec(shape, lambda i: tuple(0 for _ in shape))
    return pl.pallas_call(
        body,
        grid=(NB,),
        in_specs=[
            pl.BlockSpec((BN_BLK, D), lambda i: (i, 0)),
            pl.BlockSpec((BN_BLK, D), lambda i: (i, 0)),
            pl.BlockSpec((3, NC, BN_BLK, D), lambda i: (0, 0, i, 0)),
            pl.BlockSpec((1, 1, BN_BLK), lambda i: (i, 0, 0)),
            full((4, D, D)),
            full((5, 4, D)),
            full((D, H)),
            full((5, H)),
            full((H, C)),
            full((5, C)),
        ],
        out_specs=pl.BlockSpec((G, C), lambda i: (0, 0)),
        out_shape=jax.ShapeDtypeStruct((G, C), jnp.float32),
        scratch_shapes=[
            pltpu.VMEM((5, G, D), jnp.float32),
            pltpu.VMEM((1, G), jnp.float32),
        ],
    )(x, y0, qb, batch_r, mlp_W, mlp_vec, fc1_W, fc1_vec, fc2_W, fc2_vec)


@functools.partial(jax.jit, static_argnums=())
def kernel(x, edge_index, scatter_edge_index_0, scatter_edge_attr_0,
           scatter_edge_index_1, scatter_edge_attr_1, scatter_edge_index_2,
           scatter_edge_attr_2, scatter_edge_index_3, scatter_edge_attr_3,
           batch, mlp_W, mlp_b, mlp_bn_gamma, mlp_bn_beta, mlp_bn_mean,
           mlp_bn_var, fc1_W, fc1_b, fc1_bn_gamma, fc1_bn_beta, fc1_bn_mean,
           fc1_bn_var, fc2_W, fc2_b, fc2_bn_gamma, fc2_bn_beta, fc2_bn_mean,
           fc2_bn_var):
    del edge_index
    xp = jnp.pad(x, ((0, NP_ - N), (0, 0)))
    pad_i = ((jnp.arange(E_PAD - E, dtype=jnp.int32) * 37) % N)
    pad_e = lambda a: jnp.concatenate([a.astype(jnp.int32), pad_i])
    r4 = lambda a: pad_e(a).reshape(NC, NS, CH, CK)
    idx = [scatter_edge_index_1, scatter_edge_index_2, scatter_edge_index_3,
           scatter_edge_index_0]
    att = [scatter_edge_attr_1, scatter_edge_attr_2, scatter_edge_attr_3,
           scatter_edge_attr_0]
    sidx = [r4(a[0]) for a in idx]
    didx = [r4(a[1]) for a in idx]
    zpad = jnp.zeros((E_PAD - E,), jnp.float32)
    attr = [jnp.concatenate([a, zpad]).reshape(NC, NS, EPT) for a in att]

    sc_a = _sc_solo_passes(1, 4, ((0, 0), (0, 1), (0, 2), (0, 3)))
    pa = sc_a(xp, *sidx, *didx, *attr)

    xh1, xh2, xh3, y0 = _tc_combine(pa)

    sc_b = _sc_solo_passes(3, 1, ((0, 0), (1, 0), (2, 0)))
    qb = sc_b(xh1, xh2, xh3, sidx[3], didx[3], attr[3])

    batch_p = jnp.pad(batch.astype(jnp.int32), (0, NP_ - N),
                      constant_values=-1)
    batch_r = batch_p.reshape(NB, 1, BN_BLK)
    mlp_vec = jnp.stack([mlp_b, mlp_bn_gamma, mlp_bn_beta, mlp_bn_mean, mlp_bn_var])
    fc1_vec = jnp.stack([fc1_b, fc1_bn_gamma, fc1_bn_beta, fc1_bn_mean, fc1_bn_var])
    fc2_vec = jnp.stack([fc2_b, fc2_bn_gamma, fc2_bn_beta, fc2_bn_mean, fc2_bn_var])
    return _tc_final(xp, y0, qb, batch_r, mlp_W, mlp_vec, fc1_W, fc1_vec,
                     fc2_W, fc2_vec)


# speedup vs baseline: 1.1284x; 1.0076x over previous
import functools

import jax
import jax.numpy as jnp
from jax import lax
from jax.experimental import pallas as pl
from jax.experimental.pallas import tpu as pltpu
from jax.experimental.pallas import tpu_sc as plsc

N = 10000
NP_ = 10240
E = 320000
D = 128
G = 16
NC = 2
NS = 16
NW = NC * NS
CK = 80
CH = 128
EPT = CH * CK
E_PAD = NW * EPT
RPT = NP_ // NS
RZB = 16
SB = 16
NF = D // 16

BN_BLK = 1024
NB = NP_ // BN_BLK


def _sc_solo_passes(num_srcs, num_idx, passes):
    np_ = len(passes)
    mesh = plsc.VectorSubcoreMesh(core_axis_name="c", subcore_axis_name="s",
                                  num_cores=NC, num_subcores=NS)

    def body(*refs):
        srcs = refs[:num_srcs]
        sidx = refs[num_srcs:num_srcs + num_idx]
        didx = refs[num_srcs + num_idx:num_srcs + 2 * num_idx]
        attr = refs[num_srcs + 2 * num_idx:num_srcs + 3 * num_idx]
        out = refs[num_srcs + 3 * num_idx]
        (acc, sidx_v, didx_v, attr_v, gbuf0, gbuf1, sbuf0, sbuf1, zbuf_v,
         gsem0, gsem1, ssem0, ssem1) = refs[num_srcs + 3 * num_idx + 1:]
        gbuf = (gbuf0, gbuf1)
        sbuf = (sbuf0, sbuf1)
        gsems = (gsem0, gsem1)
        ssems = (ssem0, ssem1)
        c = lax.axis_index("c")
        s = lax.axis_index("s")
        base = s * RPT

        def zrow(r, carry):
            for f in range(NF):
                zbuf_v[r, pl.ds(f * 16, 16)] = jnp.zeros((16,), jnp.float32)
            return carry
        lax.fori_loop(0, RZB, zrow, 0)

        for p, (si, ii) in enumerate(passes):
            for k in range(RPT // RZB):
                pltpu.async_copy(
                    zbuf_v, acc.at[pl.ds(base + k * RZB, RZB), :], ssems[0])
            for k in range(RPT // RZB):
                pltpu.make_async_copy(
                    zbuf_v, acc.at[pl.ds(base + k * RZB, RZB), :],
                    ssems[0]).wait()
            plsc.subcore_barrier()
            src = srcs[si]

            dn = lax.GatherDimensionNumbers(
                offset_dims=(), collapsed_slice_dims=(0,), start_index_map=(0,))

            def scale_rows(gv, sv, j):
                @plsc.parallel_loop(0, CK // 16, 1, unroll=2)
                def _group16(gg):
                    a16 = attr_v[pl.ds(j * CK + gg * 16, 16)]
                    for k in range(16):
                        av = lax.gather(
                            a16, jnp.full((16, 1), k, jnp.int32), dn, (1,),
                            mode=lax.GatherScatterMode.PROMISE_IN_BOUNDS)
                        e = gg * 16 + k
                        for f in range(NF):
                            sv[e, pl.ds(f * 16, 16)] = (
                                gv[e, pl.ds(f * 16, 16)] * av)

            NG = SB // 2

            def block(b, carry, si=si, ii=ii):
                pltpu.sync_copy(sidx[ii].at[c, s, pl.ds(b * SB, SB)], sidx_v)
                pltpu.sync_copy(didx[ii].at[c, s, pl.ds(b * SB, SB)], didx_v)
                pltpu.sync_copy(attr[ii].at[c, s, pl.ds(b * SB * CK, SB * CK)],
                                attr_v)
                for t in range(2):
                    pltpu.async_copy(srcs[si].at[sidx_v.at[t]], gbuf[t], gsems[t])

                def group(g, carry2):
                    for t in range(2):
                        j = g * 2 + t
                        pltpu.make_async_copy(
                            srcs[si].at[sidx_v.at[j]], gbuf[t], gsems[t]).wait()

                        @pl.when(g > 0)
                        def _drain(t=t, j=j):
                            pltpu.make_async_copy(
                                sbuf[t], acc.at[didx_v.at[j]], ssems[t]).wait()
                        scale_rows(gbuf[t], sbuf[t], j)

                        @pl.when(g < NG - 1)
                        def _prefetch(t=t, j=j):
                            pltpu.async_copy(
                                srcs[si].at[sidx_v.at[j + 2]], gbuf[t], gsems[t])
                        pltpu.async_copy(sbuf[t], acc.at[didx_v.at[j]],
                                         ssems[t], add=True)
                    return carry2
                lax.fori_loop(0, NG, group, 0)
                for t in range(2):
                    pltpu.make_async_copy(
                        sbuf[t], acc.at[didx_v.at[SB - 2 + t]], ssems[t]).wait()
                return carry
            lax.fori_loop(0, CH // SB, block, 0)
            plsc.subcore_barrier()
            pltpu.sync_copy(acc.at[pl.ds(base, RPT), :],
                            out.at[p, c, pl.ds(base, RPT), :])

    return pl.kernel(
        body,
        out_type=jax.ShapeDtypeStruct((np_, NC, NP_, D), jnp.float32),
        mesh=mesh,
        scratch_types=[
            pltpu.VMEM_SHARED((NP_, D), jnp.float32),
            pltpu.VMEM((SB, CK), jnp.int32),
            pltpu.VMEM((SB, CK), jnp.int32),
            pltpu.VMEM((SB * CK,), jnp.float32),
            pltpu.VMEM((CK, D), jnp.float32),
            pltpu.VMEM((CK, D), jnp.float32),
            pltpu.VMEM((CK, D), jnp.float32),
            pltpu.VMEM((CK, D), jnp.float32),
            pltpu.VMEM((RZB, D), jnp.float32),
            pltpu.SemaphoreType.DMA,
            pltpu.SemaphoreType.DMA,
            pltpu.SemaphoreType.DMA,
            pltpu.SemaphoreType.DMA,
        ],
    )


def _tc_combine(pa, do_abs):
    def body(pa_ref, o0, o1):
        for i, ref in enumerate((o0, o1)):
            v = pa_ref[i, 0] + pa_ref[i, 1]
            ref[...] = jnp.abs(v) if do_abs[i] else v

    row = jax.ShapeDtypeStruct((NP_, D), jnp.float32)
    return pl.pallas_call(
        body,
        grid=(NB,),
        in_specs=[pl.BlockSpec((2, NC, BN_BLK, D), lambda i: (0, 0, i, 0))],
        out_specs=[pl.BlockSpec((BN_BLK, D), lambda i: (i, 0))] * 2,
        out_shape=[row, row],
    )(pa)


def _elu(v):
    return jnp.where(v > 0, v, jnp.exp(jnp.minimum(v, 0.0)) - 1.0)


def _tc_final(x, y0, qb1, qb2, batch_r, mlp_W, mlp_vec, fc1_W, fc1_vec,
              fc2_W, fc2_vec):
    C = fc2_W.shape[1]
    H = fc1_W.shape[1]
    EPS = 1e-5

    def body(x_ref, y0_ref, qb1_ref, qb2_ref, b_ref, mW_ref, mv_ref, f1W_ref,
             f1v_ref, f2W_ref, f2v_ref, out_ref, acc, cnt):
        i = pl.program_id(0)

        @pl.when(i == 0)
        def _init():
            acc[...] = jnp.zeros_like(acc)
            cnt[...] = jnp.zeros_like(cnt)

        b = b_ref[0, 0, :]
        oh = (b[:, None] == lax.broadcasted_iota(jnp.int32, (BN_BLK, G), 1)
              ).astype(jnp.float32)

        def segdot(z):
            return lax.dot_general(oh, z, (((0,), (0,)), ((), ())),
                                   preferred_element_type=jnp.float32)

        acc[0] += segdot(x_ref[...])
        acc[1] += segdot(_elu(y0_ref[...]))
        for t in range(3):
            qr = qb1_ref if t < 2 else qb2_ref
            y = qr[t % 2 if t < 2 else 0, 0] + qr[t % 2 if t < 2 else 0, 1]
            acc[2 + t] += segdot(_elu(y))
        cnt[0, :] += jnp.sum(oh, axis=0)

        @pl.when(i == NB - 1)
        def _tail():
            h = acc[0]
            csum = jnp.zeros((D,), jnp.float32)
            for t in range(4):
                bvec, gam, bet, mean, var = (mv_ref[k, t] for k in range(5))
                sc = gam * lax.rsqrt(var + EPS)
                h = h + lax.dot_general(
                    acc[1 + t], mW_ref[t] * sc[None, :],
                    (((1,), (0,)), ((), ())), preferred_element_type=jnp.float32)
                csum = csum + (bvec - mean) * sc + bet
            h = h + cnt[0, :G][:, None] * csum[None, :]
            b1, g1, be1, m1, v1 = (f1v_ref[k] for k in range(5))
            s1 = g1 * lax.rsqrt(v1 + EPS)
            h1 = lax.dot_general(h, f1W_ref[...], (((1,), (0,)), ((), ())),
                                 preferred_element_type=jnp.float32)
            h1 = (h1 + b1[None, :] - m1[None, :]) * s1[None, :] + be1[None, :]
            h1 = jnp.maximum(h1, 0.0)
            b2, g2, be2, m2, v2 = (f2v_ref[k] for k in range(5))
            s2 = g2 * lax.rsqrt(v2 + EPS)
            o = lax.dot_general(h1, f2W_ref[...], (((1,), (0,)), ((), ())),
                                preferred_element_type=jnp.float32)
            out_ref[...] = (o + b2[None, :] - m2[None, :]) * s2[None, :] + be2[None, :]

    full = lambda shape: pl.BlockSpec(shape, lambda i: tuple(0 for _ in shape))
    return pl.pallas_call(
        body,
        grid=(NB,),
        in_specs=[
            pl.BlockSpec((BN_BLK, D), lambda i: (i, 0)),
            pl.BlockSpec((BN_BLK, D), lambda i: (i, 0)),
            pl.BlockSpec((2, NC, BN_BLK, D), lambda i: (0, 0, i, 0)),
            pl.BlockSpec((1, NC, BN_BLK, D), lambda i: (0, 0, i, 0)),
            pl.BlockSpec((1, 1, BN_BLK), lambda i: (i, 0, 0)),
            full((4, D, D)),
            full((5, 4, D)),
            full((D, H)),
            full((5, H)),
            full((H, C)),
            full((5, C)),
        ],
        out_specs=pl.BlockSpec((G, C), lambda i: (0, 0)),
        out_shape=jax.ShapeDtypeStruct((G, C), jnp.float32),
        scratch_shapes=[
            pltpu.VMEM((5, G, D), jnp.float32),
            pltpu.VMEM((1, G), jnp.float32),
        ],
    )(x, y0, qb1, qb2, batch_r, mlp_W, mlp_vec, fc1_W, fc1_vec, fc2_W,
      fc2_vec)


@functools.partial(jax.jit, static_argnums=())
def kernel(x, edge_index, scatter_edge_index_0, scatter_edge_attr_0,
           scatter_edge_index_1, scatter_edge_attr_1, scatter_edge_index_2,
           scatter_edge_attr_2, scatter_edge_index_3, scatter_edge_attr_3,
           batch, mlp_W, mlp_b, mlp_bn_gamma, mlp_bn_beta, mlp_bn_mean,
           mlp_bn_var, fc1_W, fc1_b, fc1_bn_gamma, fc1_bn_beta, fc1_bn_mean,
           fc1_bn_var, fc2_W, fc2_b, fc2_bn_gamma, fc2_bn_beta, fc2_bn_mean,
           fc2_bn_var):
    del edge_index
    xp = jnp.pad(x, ((0, NP_ - N), (0, 0)))
    pad_i = ((jnp.arange(E_PAD - E, dtype=jnp.int32) * 37) % N)
    pad_e = lambda a: jnp.concatenate([a.astype(jnp.int32), pad_i])
    r4 = lambda a: pad_e(a).reshape(NC, NS, CH, CK)
    idx = [scatter_edge_index_1, scatter_edge_index_2, scatter_edge_index_3,
           scatter_edge_index_0]
    att = [scatter_edge_attr_1, scatter_edge_attr_2, scatter_edge_attr_3,
           scatter_edge_attr_0]
    sidx = [r4(a[0]) for a in idx]
    didx = [r4(a[1]) for a in idx]
    zpad = jnp.zeros((E_PAD - E,), jnp.float32)
    attr = [jnp.concatenate([a, zpad]).reshape(NC, NS, EPT) for a in att]

    sc_a2 = _sc_solo_passes(1, 2, ((0, 0), (0, 1)))
    pa1 = sc_a2(xp, sidx[0], sidx[1], didx[0], didx[1], attr[0], attr[1])
    pa2 = sc_a2(xp, sidx[2], sidx[3], didx[2], didx[3], attr[2], attr[3])
    xh1, xh2 = _tc_combine(pa1, (True, True))

    sc_b2 = _sc_solo_passes(2, 1, ((0, 0), (1, 0)))
    qb1 = sc_b2(xh1, xh2, sidx[3], didx[3], attr[3])
    xh3, y0 = _tc_combine(pa2, (True, False))
    sc_b1 = _sc_solo_passes(1, 1, ((0, 0),))
    qb2 = sc_b1(xh3, sidx[3], didx[3], attr[3])

    batch_p = jnp.pad(batch.astype(jnp.int32), (0, NP_ - N),
                      constant_values=-1)
    batch_r = batch_p.reshape(NB, 1, BN_BLK)
    mlp_vec = jnp.stack([mlp_b, mlp_bn_gamma, mlp_bn_beta, mlp_bn_mean, mlp_bn_var])
    fc1_vec = jnp.stack([fc1_b, fc1_bn_gamma, fc1_bn_beta, fc1_bn_mean, fc1_bn_var])
    fc2_vec = jnp.stack([fc2_b, fc2_bn_gamma, fc2_bn_beta, fc2_bn_mean, fc2_bn_var])
    return _tc_final(xp, y0, qb1, qb2, batch_r, mlp_W, mlp_vec, fc1_W,
                     fc1_vec, fc2_W, fc2_vec)


# speedup vs baseline: 1.1914x; 1.0559x over previous
import functools

import jax
import jax.numpy as jnp
from jax import lax
from jax.experimental import pallas as pl
from jax.experimental.pallas import tpu as pltpu
from jax.experimental.pallas import tpu_sc as plsc

N = 10000
NP_ = 10240
E = 320000
D = 128
G = 16
NC = 2
NS = 16
NW = NC * NS
CK = 80
CH = 128
EPT = CH * CK
E_PAD = NW * EPT
RPT = NP_ // NS
RZB = 16
SB = 16
NF = D // 16

BN_BLK = 1024
NB = NP_ // BN_BLK


def _sc_solo_passes(num_srcs, num_idx, passes):
    np_ = len(passes)
    mesh = plsc.VectorSubcoreMesh(core_axis_name="c", subcore_axis_name="s",
                                  num_cores=NC, num_subcores=NS)

    def body(*refs):
        srcs = refs[:num_srcs]
        sidx = refs[num_srcs:num_srcs + num_idx]
        didx = refs[num_srcs + num_idx:num_srcs + 2 * num_idx]
        attr = refs[num_srcs + 2 * num_idx:num_srcs + 3 * num_idx]
        out = refs[num_srcs + 3 * num_idx]
        (acc, sidx_v, didx_v, attr_v, gbuf0, gbuf1, sbuf0, sbuf1, zbuf_v,
         gsem0, gsem1, ssem0, ssem1) = refs[num_srcs + 3 * num_idx + 1:]
        gbuf = (gbuf0, gbuf1)
        sbuf = (sbuf0, sbuf1)
        gsems = (gsem0, gsem1)
        ssems = (ssem0, ssem1)
        c = lax.axis_index("c")
        s = lax.axis_index("s")
        base = s * RPT

        def zrow(r, carry):
            for f in range(NF):
                zbuf_v[r, pl.ds(f * 16, 16)] = jnp.zeros((16,), jnp.float32)
            return carry
        lax.fori_loop(0, RZB, zrow, 0)

        for p, (si, ii) in enumerate(passes):
            for k in range(RPT // RZB):
                pltpu.async_copy(
                    zbuf_v, acc.at[pl.ds(base + k * RZB, RZB), :], ssems[0])
            for k in range(RPT // RZB):
                pltpu.make_async_copy(
                    zbuf_v, acc.at[pl.ds(base + k * RZB, RZB), :],
                    ssems[0]).wait()
            plsc.subcore_barrier()
            src = srcs[si]

            dn = lax.GatherDimensionNumbers(
                offset_dims=(), collapsed_slice_dims=(0,), start_index_map=(0,))

            def scale_rows(gv, sv, j):
                @plsc.parallel_loop(0, CK // 16, 1, unroll=2)
                def _group16(gg):
                    a16 = attr_v[pl.ds(j * CK + gg * 16, 16)]
                    for k in range(16):
                        av = lax.gather(
                            a16, jnp.full((16, 1), k, jnp.int32), dn, (1,),
                            mode=lax.GatherScatterMode.PROMISE_IN_BOUNDS)
                        e = gg * 16 + k
                        for f in range(NF):
                            sv[e, pl.ds(f * 16, 16)] = (
                                gv[e, pl.ds(f * 16, 16)] * av)

            NG = SB // 2

            def block(b, carry, si=si, ii=ii):
                pltpu.async_copy(sidx[ii].at[c, s, pl.ds(b * SB, SB)], sidx_v,
                                 ssems[0])
                pltpu.async_copy(didx[ii].at[c, s, pl.ds(b * SB, SB)], didx_v,
                                 ssems[0])
                pltpu.async_copy(attr[ii].at[c, s, pl.ds(b * SB * CK, SB * CK)],
                                 attr_v, ssems[0])
                pltpu.make_async_copy(
                    sidx[ii].at[c, s, pl.ds(b * SB, SB)], sidx_v, ssems[0]).wait()
                pltpu.make_async_copy(
                    didx[ii].at[c, s, pl.ds(b * SB, SB)], didx_v, ssems[0]).wait()
                pltpu.make_async_copy(
                    attr[ii].at[c, s, pl.ds(b * SB * CK, SB * CK)], attr_v,
                    ssems[0]).wait()
                for t in range(2):
                    pltpu.async_copy(srcs[si].at[sidx_v.at[t]], gbuf[t], gsems[t])

                def group(g, carry2):
                    for t in range(2):
                        j = g * 2 + t
                        pltpu.make_async_copy(
                            srcs[si].at[sidx_v.at[j]], gbuf[t], gsems[t]).wait()

                        @pl.when(g > 0)
                        def _drain(t=t, j=j):
                            pltpu.make_async_copy(
                                sbuf[t], acc.at[didx_v.at[j]], ssems[t]).wait()
                        scale_rows(gbuf[t], sbuf[t], j)

                        @pl.when(g < NG - 1)
                        def _prefetch(t=t, j=j):
                            pltpu.async_copy(
                                srcs[si].at[sidx_v.at[j + 2]], gbuf[t], gsems[t])
                        pltpu.async_copy(sbuf[t], acc.at[didx_v.at[j]],
                                         ssems[t], add=True)
                    return carry2
                lax.fori_loop(0, NG, group, 0)
                for t in range(2):
                    pltpu.make_async_copy(
                        sbuf[t], acc.at[didx_v.at[SB - 2 + t]], ssems[t]).wait()
                return carry
            lax.fori_loop(0, CH // SB, block, 0)
            plsc.subcore_barrier()
            pltpu.sync_copy(acc.at[pl.ds(base, RPT), :],
                            out.at[p, c, pl.ds(base, RPT), :])

    return pl.kernel(
        body,
        out_type=jax.ShapeDtypeStruct((np_, NC, NP_, D), jnp.float32),
        mesh=mesh,
        scratch_types=[
            pltpu.VMEM_SHARED((NP_, D), jnp.float32),
            pltpu.VMEM((SB, CK), jnp.int32),
            pltpu.VMEM((SB, CK), jnp.int32),
            pltpu.VMEM((SB * CK,), jnp.float32),
            pltpu.VMEM((CK, D), jnp.float32),
            pltpu.VMEM((CK, D), jnp.float32),
            pltpu.VMEM((CK, D), jnp.float32),
            pltpu.VMEM((CK, D), jnp.float32),
            pltpu.VMEM((RZB, D), jnp.float32),
            pltpu.SemaphoreType.DMA,
            pltpu.SemaphoreType.DMA,
            pltpu.SemaphoreType.DMA,
            pltpu.SemaphoreType.DMA,
        ],
    )


def _tc_combine(pa, do_abs):
    def body(pa_ref, o0, o1):
        for i, ref in enumerate((o0, o1)):
            v = pa_ref[i, 0] + pa_ref[i, 1]
            ref[...] = jnp.abs(v) if do_abs[i] else v

    row = jax.ShapeDtypeStruct((NP_, D), jnp.float32)
    return pl.pallas_call(
        body,
        grid=(NB,),
        in_specs=[pl.BlockSpec((2, NC, BN_BLK, D), lambda i: (0, 0, i, 0))],
        out_specs=[pl.BlockSpec((BN_BLK, D), lambda i: (i, 0))] * 2,
        out_shape=[row, row],
    )(pa)


def _elu(v):
    return jnp.where(v > 0, v, jnp.exp(jnp.minimum(v, 0.0)) - 1.0)


def _tc_final(x, y0, qb1, qb2, batch_r, mlp_W, mlp_vec, fc1_W, fc1_vec,
              fc2_W, fc2_vec):
    C = fc2_W.shape[1]
    H = fc1_W.shape[1]
    EPS = 1e-5

    def body(x_ref, y0_ref, qb1_ref, qb2_ref, b_ref, mW_ref, mv_ref, f1W_ref,
             f1v_ref, f2W_ref, f2v_ref, out_ref, acc, cnt):
        i = pl.program_id(0)

        @pl.when(i == 0)
        def _init():
            acc[...] = jnp.zeros_like(acc)
            cnt[...] = jnp.zeros_like(cnt)

        b = b_ref[0, 0, :]
        oh = (b[:, None] == lax.broadcasted_iota(jnp.int32, (BN_BLK, G), 1)
              ).astype(jnp.float32)

        def segdot(z):
            return lax.dot_general(oh, z, (((0,), (0,)), ((), ())),
                                   preferred_element_type=jnp.float32)

        acc[0] += segdot(x_ref[...])
        acc[1] += segdot(_elu(y0_ref[...]))
        for t in range(3):
            qr = qb1_ref if t < 2 else qb2_ref
            y = qr[t % 2 if t < 2 else 0, 0] + qr[t % 2 if t < 2 else 0, 1]
            acc[2 + t] += segdot(_elu(y))
        cnt[0, :] += jnp.sum(oh, axis=0)

        @pl.when(i == NB - 1)
        def _tail():
            h = acc[0]
            csum = jnp.zeros((D,), jnp.float32)
            for t in range(4):
                bvec, gam, bet, mean, var = (mv_ref[k, t] for k in range(5))
                sc = gam * lax.rsqrt(var + EPS)
                h = h + lax.dot_general(
                    acc[1 + t], mW_ref[t] * sc[None, :],
                    (((1,), (0,)), ((), ())), preferred_element_type=jnp.float32)
                csum = csum + (bvec - mean) * sc + bet
            h = h + cnt[0, :G][:, None] * csum[None, :]
            b1, g1, be1, m1, v1 = (f1v_ref[k] for k in range(5))
            s1 = g1 * lax.rsqrt(v1 + EPS)
            h1 = lax.dot_general(h, f1W_ref[...], (((1,), (0,)), ((), ())),
                                 preferred_element_type=jnp.float32)
            h1 = (h1 + b1[None, :] - m1[None, :]) * s1[None, :] + be1[None, :]
            h1 = jnp.maximum(h1, 0.0)
            b2, g2, be2, m2, v2 = (f2v_ref[k] for k in range(5))
            s2 = g2 * lax.rsqrt(v2 + EPS)
            o = lax.dot_general(h1, f2W_ref[...], (((1,), (0,)), ((), ())),
                                preferred_element_type=jnp.float32)
            out_ref[...] = (o + b2[None, :] - m2[None, :]) * s2[None, :] + be2[None, :]

    full = lambda shape: pl.BlockSpec(shape, lambda i: tuple(0 for _ in shape))
    return pl.pallas_call(
        body,
        grid=(NB,),
        in_specs=[
            pl.BlockSpec((BN_BLK, D), lambda i: (i, 0)),
            pl.BlockSpec((BN_BLK, D), lambda i: (i, 0)),
            pl.BlockSpec((2, NC, BN_BLK, D), lambda i: (0, 0, i, 0)),
            pl.BlockSpec((1, NC, BN_BLK, D), lambda i: (0, 0, i, 0)),
            pl.BlockSpec((1, 1, BN_BLK), lambda i: (i, 0, 0)),
            full((4, D, D)),
            full((5, 4, D)),
            full((D, H)),
            full((5, H)),
            full((H, C)),
            full((5, C)),
        ],
        out_specs=pl.BlockSpec((G, C), lambda i: (0, 0)),
        out_shape=jax.ShapeDtypeStruct((G, C), jnp.float32),
        scratch_shapes=[
            pltpu.VMEM((5, G, D), jnp.float32),
            pltpu.VMEM((1, G), jnp.float32),
        ],
    )(x, y0, qb1, qb2, batch_r, mlp_W, mlp_vec, fc1_W, fc1_vec, fc2_W,
      fc2_vec)


@functools.partial(jax.jit, static_argnums=())
def kernel(x, edge_index, scatter_edge_index_0, scatter_edge_attr_0,
           scatter_edge_index_1, scatter_edge_attr_1, scatter_edge_index_2,
           scatter_edge_attr_2, scatter_edge_index_3, scatter_edge_attr_3,
           batch, mlp_W, mlp_b, mlp_bn_gamma, mlp_bn_beta, mlp_bn_mean,
           mlp_bn_var, fc1_W, fc1_b, fc1_bn_gamma, fc1_bn_beta, fc1_bn_mean,
           fc1_bn_var, fc2_W, fc2_b, fc2_bn_gamma, fc2_bn_beta, fc2_bn_mean,
           fc2_bn_var):
    del edge_index
    xp = jnp.pad(x, ((0, NP_ - N), (0, 0)))
    pad_i = ((jnp.arange(E_PAD - E, dtype=jnp.int32) * 37) % N)
    pad_e = lambda a: jnp.concatenate([a.astype(jnp.int32), pad_i])
    r4 = lambda a: pad_e(a).reshape(NC, NS, CH, CK)
    idx = [scatter_edge_index_1, scatter_edge_index_2, scatter_edge_index_3,
           scatter_edge_index_0]
    att = [scatter_edge_attr_1, scatter_edge_attr_2, scatter_edge_attr_3,
           scatter_edge_attr_0]
    sidx = [r4(a[0]) for a in idx]
    didx = [r4(a[1]) for a in idx]
    zpad = jnp.zeros((E_PAD - E,), jnp.float32)
    attr = [jnp.concatenate([a, zpad]).reshape(NC, NS, EPT) for a in att]

    sc_a2 = _sc_solo_passes(1, 2, ((0, 0), (0, 1)))
    pa1 = sc_a2(xp, sidx[0], sidx[1], didx[0], didx[1], attr[0], attr[1])
    pa2 = sc_a2(xp, sidx[2], sidx[3], didx[2], didx[3], attr[2], attr[3])
    xh1, xh2 = _tc_combine(pa1, (True, True))

    sc_b2 = _sc_solo_passes(2, 1, ((0, 0), (1, 0)))
    qb1 = sc_b2(xh1, xh2, sidx[3], didx[3], attr[3])
    xh3, y0 = _tc_combine(pa2, (True, False))
    sc_b1 = _sc_solo_passes(1, 1, ((0, 0),))
    qb2 = sc_b1(xh3, sidx[3], didx[3], attr[3])

    batch_p = jnp.pad(batch.astype(jnp.int32), (0, NP_ - N),
                      constant_values=-1)
    batch_r = batch_p.reshape(NB, 1, BN_BLK)
    mlp_vec = jnp.stack([mlp_b, mlp_bn_gamma, mlp_bn_beta, mlp_bn_mean, mlp_bn_var])
    fc1_vec = jnp.stack([fc1_b, fc1_bn_gamma, fc1_bn_beta, fc1_bn_mean, fc1_bn_var])
    fc2_vec = jnp.stack([fc2_b, fc2_bn_gamma, fc2_bn_beta, fc2_bn_mean, fc2_bn_var])
    return _tc_final(xp, y0, qb1, qb2, batch_r, mlp_W, mlp_vec, fc1_W,
                     fc1_vec, fc2_W, fc2_vec)
